# baseline (speedup 1.0000x reference)
import functools

import jax
import jax.numpy as jnp
from jax import lax
from jax.experimental import pallas as pl
from jax.experimental.pallas import tpu as pltpu
from jax.experimental.pallas import tpu_sc as plsc

N_USER = 10000
N_ITEM = 10000
N = 10000
D = 128
E = 160000

NT = 16
EPT = E // NT
CHUNK = 80
NCHUNK = EPT // CHUNK
ROWS_PT = N // NT
N_S = 10240
NCHUNK_PAD = 128
EPT_PAD = NCHUNK_PAD * CHUNK


def _pre_body(xu, xi, n0, n1, wlu, wru, wli, wri, blu, bru, bli, bri,
              hl_u2i, hr_u2i, hl_i2u, hr_i2u):
    a = xu[...] + n0[...]
    b = xi[...] + n1[...]
    hl_u2i[...] = jnp.dot(a, wlu[...], preferred_element_type=jnp.float32) + blu[...]
    hr_u2i[...] = jnp.dot(b, wru[...], preferred_element_type=jnp.float32) + bru[...]
    hl_i2u[...] = jnp.dot(b, wli[...], preferred_element_type=jnp.float32) + bli[...]
    hr_i2u[...] = jnp.dot(a, wri[...], preferred_element_type=jnp.float32) + bri[...]


def _dense_pre(xu, xi, n0, n1, p):
    blk = 1000
    grid = (N // blk,)
    row = pl.BlockSpec((blk, D), lambda i: (i, 0))
    full = pl.BlockSpec((D, D), lambda i: (0, 0))
    vec = pl.BlockSpec((1, D), lambda i: (0, 0))
    return pl.pallas_call(
        _pre_body,
        grid=grid,
        in_specs=[row, row, row, row, full, full, full, full, vec, vec, vec, vec],
        out_specs=[row, row, row, row],
        out_shape=[jax.ShapeDtypeStruct((N, D), jnp.float32)] * 4,
    )(xu, xi, n0, n1,
      p['Wl_u2i'], p['Wr_u2i'], p['Wl_i2u'], p['Wr_i2u'],
      p['bl_u2i'].reshape(1, D), p['br_u2i'].reshape(1, D),
      p['bl_i2u'].reshape(1, D), p['br_i2u'].reshape(1, D))


def _attr_body(xh, w1, b1, g1, be1, w2, b2, g2, be2, w3, b3, g3, be3,
               wmu, bmu, wlv, blv, a_mu, a_lv):
    h = xh[...]
    for (w, b, g, be) in ((w1, b1, g1, be1), (w2, b2, g2, be2), (w3, b3, g3, be3)):
        h = jnp.tanh(jnp.dot(h, w[...], preferred_element_type=jnp.float32) + b[...])
        mean = jnp.mean(h, axis=0, keepdims=True)
        var = jnp.mean((h - mean) ** 2, axis=0, keepdims=True)
        h = (h - mean) / jnp.sqrt(var + 1e-5) * g[...] + be[...]
    a_mu[...] = jnp.dot(h, wmu[...], preferred_element_type=jnp.float32) + bmu[...]
    a_lv[...] = jnp.dot(h, wlv[...], preferred_element_type=jnp.float32) + blv[...]


def _attr_branch(x_hat, p):
    n = x_hat.shape[0]
    args = [x_hat]
    for j in range(1, 4):
        args += [p['W%d' % j], p['b%d' % j].reshape(1, D),
                 p['bn%d_g' % j].reshape(1, D), p['bn%d_b' % j].reshape(1, D)]
    args += [p['attr_mu_W'], p['attr_mu_b'].reshape(1, D),
             p['attr_logvar_W'], p['attr_logvar_b'].reshape(1, D)]
    return pl.pallas_call(
        _attr_body,
        out_shape=[jax.ShapeDtypeStruct((n, D), jnp.float32)] * 2,
    )(*args)


def _post_body(oi, ou, bi, bu, wmu, bmu, wlv, blv,
               user_mu, user_lv, item_mu, item_lv):
    h_item = jnp.maximum(oi[...] + bi[...], 0.0)
    h_user = jnp.maximum(ou[...] + bu[...], 0.0)
    user_mu[...] = jnp.dot(h_user, wmu[...], preferred_element_type=jnp.float32) + bmu[...]
    user_lv[...] = jnp.dot(h_user, wlv[...], preferred_element_type=jnp.float32) + blv[...]
    item_mu[...] = jnp.dot(h_item, wmu[...], preferred_element_type=jnp.float32) + bmu[...]
    item_lv[...] = jnp.dot(h_item, wlv[...], preferred_element_type=jnp.float32) + blv[...]


def _post(out_item, out_user, p):
    blk = 1000
    grid = (N // blk,)
    row = pl.BlockSpec((blk, D), lambda i: (i, 0))
    full = pl.BlockSpec((D, D), lambda i: (0, 0))
    vec = pl.BlockSpec((1, D), lambda i: (0, 0))
    return pl.pallas_call(
        _post_body,
        grid=grid,
        in_specs=[row, row, vec, vec, full, vec, full, vec],
        out_specs=[row, row, row, row],
        out_shape=[jax.ShapeDtypeStruct((N, D), jnp.float32)] * 4,
    )(out_item, out_user,
      p['bias_u2i'].reshape(1, D), p['bias_i2u'].reshape(1, D),
      p['node_mu_W'], p['node_mu_b'].reshape(1, D),
      p['node_logvar_W'], p['node_logvar_b'].reshape(1, D))


def _edge_body(hl_hbm, hr_hbm, src4_hbm, dst4_hbm, att_hbm, out_hbm,
               srcb, dstb, ex_v, sg_v, rowA0, rowA1, rowB,
               att_v, zb_v, s_sh, out_sh, semA0, semA1, semB, semS):
    c = lax.axis_index("c")
    w = lax.axis_index("s")
    voff = pl.multiple_of(c * N, 8)
    hlv = hl_hbm.at[pl.ds(voff, N)]
    hrv = hr_hbm.at[pl.ds(voff, N)]

    zv = jnp.zeros((16,), jnp.float32)

    def _zero_zb(i, _):
        zb_v[pl.ds(i * 16, 16)] = zv
        return 0
    lax.fori_loop(0, 40, _zero_zb, 0)

    def _zero_rowA(i, _):
        r = i // 8
        k = i % 8
        rowA0[r, pl.ds(k * 16, 16)] = zv
        return 0
    lax.fori_loop(0, CHUNK * 8, _zero_rowA, 0)

    pltpu.sync_copy(zb_v, s_sh.at[pl.ds(w * 640, 640)])

    rstart = w * 624
    nrow16 = jnp.where(w == NT - 1, 40, 39)

    def _zero_out(i, _):
        pltpu.sync_copy(rowA0.at[pl.ds(0, 16)], out_sh.at[pl.ds(rstart + i * 16, 16)])
        return 0
    lax.fori_loop(0, nrow16, _zero_out, 0)

    pltpu.sync_copy(att_hbm.at[pl.ds(c * D, D)], att_v)

    plsc.subcore_barrier()

    att_regs = [att_v[pl.ds(k * 16, 16)] for k in range(8)]
    iota16 = lax.iota(jnp.int32, 16)

    rowAs = (rowA0, rowA1)
    semAs = (semA0, semA1)

    NBLK = NCHUNK_PAD // 32
    BLKC = 32

    def _refill(blk):
        pltpu.sync_copy(src4_hbm.at[c, w, pl.ds(blk * BLKC, BLKC)], srcb)
        pltpu.sync_copy(dst4_hbm.at[c, w, pl.ds(blk * BLKC, BLKC)], dstb)

    def _issueA(i, p):
        pltpu.async_copy(hlv.at[srcb.at[i]], rowAs[p], semAs[p])

    def _waitA(i, p):
        pltpu.make_async_copy(hlv.at[srcb.at[i]], rowAs[p], semAs[p]).wait()

    def _run_blocks(compute, issue_side, block_end=None):
        def _blk_body(blk, _):
            _refill(blk)
            _issueA(0, 0)
            issue_side(blk, 0, 0)

            def _body(j, _):
                a = 2 * j + 1
                b = 2 * j + 2
                _issueA(a, 1)
                compute(blk, 2 * j, 0)
                issue_side(blk, a, 1)
                _issueA(b, 0)
                compute(blk, a, 1)
                issue_side(blk, b, 0)
                return 0
            lax.fori_loop(0, BLKC // 2 - 1, _body, 0)
            _issueA(BLKC - 1, 1)
            compute(blk, BLKC - 2, 0)
            issue_side(blk, BLKC - 1, 1)
            compute(blk, BLKC - 1, 1)
            if block_end is not None:
                block_end(blk)
            return 0
        lax.fori_loop(0, NBLK, _blk_body, 0)

    def _issueB(blk, i, p):
        pltpu.async_copy(hrv.at[dstb.at[i]], rowB, semB)

    def _compute1(blk, i, p):
        _waitA(i, p)
        pltpu.make_async_copy(hrv.at[dstb.at[i]], rowB, semB).wait()
        ebase = blk * BLKC * CHUNK + i * CHUNK
        rA = rowAs[p]

        def _group(g, _):
            gb = g * 16
            evec = jnp.zeros((16,), jnp.float32)
            for e in range(16):
                acc = jnp.zeros((16,), jnp.float32)
                for k in range(8):
                    sl = pl.ds(k * 16, 16)
                    t = rA[gb + e, sl] + rowB[gb + e, sl]
                    z = jnp.maximum(t, 0.2 * t)
                    acc = acc + z * att_regs[k]
                lanes = [acc[i2] for i2 in range(16)]
                while len(lanes) > 1:
                    lanes = [lanes[i2] + lanes[i2 + 1] for i2 in range(0, len(lanes), 2)]
                evec = jnp.where(iota16 == e, lanes[0], evec)
            ex_v[pl.ds(ebase + gb, 16)] = jnp.exp(evec)
            return 0
        lax.fori_loop(0, CHUNK // 16, _group, 0)

        pltpu.sync_copy(ex_v.at[pl.ds(ebase, CHUNK)], s_sh.at[dstb.at[i]], add=True)

    def _drain_block(blk):
        def _d(i, _):
            ebase = blk * BLKC * CHUNK + i * CHUNK
            pltpu.make_async_copy(ex_v.at[pl.ds(ebase, CHUNK)],
                                  s_sh.at[dstb.at[i]], semS).wait()
            return 0
        lax.fori_loop(0, BLKC, _d, 0)

    _sc1 = jax.named_scope("edge_phase1")
    _sc1.__enter__()
    _run_blocks(_compute1, _issueB)
    _sc1.__exit__(None, None, None)

    plsc.subcore_barrier()

    def _issueS(blk, i, p):
        pltpu.async_copy(s_sh.at[dstb.at[i]], sg_v, semB)

    def _compute2(blk, i, p):
        _waitA(i, p)
        pltpu.make_async_copy(s_sh.at[dstb.at[i]], sg_v, semB).wait()
        ebase = blk * BLKC * CHUNK + i * CHUNK
        rA = rowAs[p]

        def _alpha(g, _):
            sl = pl.ds(ebase + g * 16, 16)
            sv = sg_v[pl.ds(g * 16, 16)]
            ex_v[sl] = ex_v[sl] / (sv + 1e-16)
            return 0
        lax.fori_loop(0, CHUNK // 16, _alpha, 0)

        def _scale(g, _):
            gb = g * 16
            av16 = ex_v[pl.ds(ebase + gb, 16)]
            for e in range(16):
                av = jnp.broadcast_to(av16[e], (16,))
                for k in range(8):
                    sl = pl.ds(k * 16, 16)
                    rA[gb + e, sl] = rA[gb + e, sl] * av
            return 0
        lax.fori_loop(0, CHUNK // 16, _scale, 0)

        pltpu.sync_copy(rA, out_sh.at[dstb.at[i]], add=True)

    _sc2 = jax.named_scope("edge_phase2")
    _sc2.__enter__()
    _run_blocks(_compute2, _issueS)
    _sc2.__exit__(None, None, None)

    plsc.subcore_barrier()

    def _wb(i, _):
        r = rstart + i * 16
        pltpu.sync_copy(out_sh.at[pl.ds(r, 16)], rowA0.at[pl.ds(0, 16)])
        pltpu.sync_copy(rowA0.at[pl.ds(0, 16)], out_hbm.at[c, pl.ds(r, 16)])
        return 0
    lax.fori_loop(0, nrow16, _wb, 0)


def _edge_stage(hl_all, hr_all, src4, dst4, att):
    mesh = plsc.VectorSubcoreMesh(core_axis_name="c", subcore_axis_name="s")
    f = pl.kernel(
        _edge_body,
        out_type=jax.ShapeDtypeStruct((2, N, D), jnp.float32),
        mesh=mesh,
        scratch_types=[
            pltpu.VMEM((32, CHUNK), jnp.int32),
            pltpu.VMEM((32, CHUNK), jnp.int32),
            pltpu.VMEM((EPT_PAD,), jnp.float32),
            pltpu.VMEM((CHUNK,), jnp.float32),
            pltpu.VMEM((CHUNK, D), jnp.float32),
            pltpu.VMEM((CHUNK, D), jnp.float32),
            pltpu.VMEM((CHUNK, D), jnp.float32),
            pltpu.VMEM((D,), jnp.float32),
            pltpu.VMEM((640,), jnp.float32),
            pltpu.VMEM_SHARED((N_S,), jnp.float32),
            pltpu.VMEM_SHARED((N + 16, D), jnp.float32),
            pltpu.SemaphoreType.DMA,
            pltpu.SemaphoreType.DMA,
            pltpu.SemaphoreType.DMA,
            pltpu.SemaphoreType.DMA,
        ],
    )
    return f(hl_all, hr_all, src4, dst4, att)


def kernel(x_user, x_item, edge_index_u2i, edge_index_i2u, params):
    p = params
    kn = jax.random.split(jax.random.key(42), 3)
    n0 = jax.random.normal(kn[0], x_user.shape, dtype=jnp.float32)
    n1 = jax.random.normal(kn[1], x_item.shape, dtype=jnp.float32)
    n2 = jax.random.normal(kn[2], (N_USER + N_ITEM, D), dtype=jnp.float32)

    hl_u2i, hr_u2i, hl_i2u, hr_i2u = _dense_pre(x_user, x_item, n0, n1, p)

    hl_all = jnp.concatenate([hl_u2i, hl_i2u], axis=0)
    hr_all = jnp.concatenate([hr_u2i, hr_i2u], axis=0)
    src = jnp.concatenate([edge_index_u2i[0], edge_index_i2u[0]])
    dst = jnp.concatenate([edge_index_u2i[1], edge_index_i2u[1]])
    src4 = jnp.pad(src.reshape(2, NT, NCHUNK, CHUNK),
                   ((0, 0), (0, 0), (0, NCHUNK_PAD - NCHUNK), (0, 0)))
    dst4 = jnp.pad(dst.reshape(2, NT, NCHUNK, CHUNK),
                   ((0, 0), (0, 0), (0, NCHUNK_PAD - NCHUNK), (0, 0)),
                   constant_values=N)
    att = jnp.concatenate([p['att_u2i'], p['att_i2u']])

    out = _edge_stage(hl_all, hr_all, src4, dst4, att)
    out_item, out_user = out[0], out[1]

    user_mu, user_lv, item_mu, item_lv = _post(out_item, out_user, p)

    x_hat = jnp.concatenate([x_user, x_item], axis=0) + n2
    a_mu, a_lv = _attr_branch(x_hat, p)

    return (user_mu, user_lv, item_mu, item_lv, a_mu, a_lv)


# speedup vs baseline: 1.5023x; 1.5023x over previous
import functools

import jax
import jax.numpy as jnp
from jax import lax
from jax.experimental import pallas as pl
from jax.experimental.pallas import tpu as pltpu
from jax.experimental.pallas import tpu_sc as plsc

N_USER = 10000
N_ITEM = 10000
N = 10000
D = 128
E = 160000

NT = 16
EPT = E // NT
CHUNK = 80
NCHUNK = EPT // CHUNK
ROWS_PT = N // NT
N_S = 10240
NCHUNK_PAD = 128
EPT_PAD = NCHUNK_PAD * CHUNK


def _pre_body(xu, xi, n0, n1, wlu, wru, wli, wri, blu, bru, bli, bri,
              hl_u2i, hr_u2i, hl_i2u, hr_i2u):
    a = xu[...] + n0[...]
    b = xi[...] + n1[...]
    hl_u2i[...] = jnp.dot(a, wlu[...], preferred_element_type=jnp.float32) + blu[...]
    hr_u2i[...] = jnp.dot(b, wru[...], preferred_element_type=jnp.float32) + bru[...]
    hl_i2u[...] = jnp.dot(b, wli[...], preferred_element_type=jnp.float32) + bli[...]
    hr_i2u[...] = jnp.dot(a, wri[...], preferred_element_type=jnp.float32) + bri[...]


def _dense_pre(xu, xi, n0, n1, p):
    blk = 1000
    grid = (N // blk,)
    row = pl.BlockSpec((blk, D), lambda i: (i, 0))
    full = pl.BlockSpec((D, D), lambda i: (0, 0))
    vec = pl.BlockSpec((1, D), lambda i: (0, 0))
    return pl.pallas_call(
        _pre_body,
        grid=grid,
        in_specs=[row, row, row, row, full, full, full, full, vec, vec, vec, vec],
        out_specs=[row, row, row, row],
        out_shape=[jax.ShapeDtypeStruct((N, D), jnp.float32)] * 4,
    )(xu, xi, n0, n1,
      p['Wl_u2i'], p['Wr_u2i'], p['Wl_i2u'], p['Wr_i2u'],
      p['bl_u2i'].reshape(1, D), p['br_u2i'].reshape(1, D),
      p['bl_i2u'].reshape(1, D), p['br_i2u'].reshape(1, D))


def _attr_body(xh, w1, b1, g1, be1, w2, b2, g2, be2, w3, b3, g3, be3,
               wmu, bmu, wlv, blv, a_mu, a_lv):
    h = xh[...]
    for (w, b, g, be) in ((w1, b1, g1, be1), (w2, b2, g2, be2), (w3, b3, g3, be3)):
        h = jnp.tanh(jnp.dot(h, w[...], preferred_element_type=jnp.float32) + b[...])
        mean = jnp.mean(h, axis=0, keepdims=True)
        var = jnp.mean((h - mean) ** 2, axis=0, keepdims=True)
        h = (h - mean) / jnp.sqrt(var + 1e-5) * g[...] + be[...]
    a_mu[...] = jnp.dot(h, wmu[...], preferred_element_type=jnp.float32) + bmu[...]
    a_lv[...] = jnp.dot(h, wlv[...], preferred_element_type=jnp.float32) + blv[...]


def _attr_branch(x_hat, p):
    n = x_hat.shape[0]
    args = [x_hat]
    for j in range(1, 4):
        args += [p['W%d' % j], p['b%d' % j].reshape(1, D),
                 p['bn%d_g' % j].reshape(1, D), p['bn%d_b' % j].reshape(1, D)]
    args += [p['attr_mu_W'], p['attr_mu_b'].reshape(1, D),
             p['attr_logvar_W'], p['attr_logvar_b'].reshape(1, D)]
    return pl.pallas_call(
        _attr_body,
        out_shape=[jax.ShapeDtypeStruct((n, D), jnp.float32)] * 2,
    )(*args)


def _post_body(oi, ou, bi, bu, wmu, bmu, wlv, blv,
               user_mu, user_lv, item_mu, item_lv):
    h_item = jnp.maximum(oi[...] + bi[...], 0.0)
    h_user = jnp.maximum(ou[...] + bu[...], 0.0)
    user_mu[...] = jnp.dot(h_user, wmu[...], preferred_element_type=jnp.float32) + bmu[...]
    user_lv[...] = jnp.dot(h_user, wlv[...], preferred_element_type=jnp.float32) + blv[...]
    item_mu[...] = jnp.dot(h_item, wmu[...], preferred_element_type=jnp.float32) + bmu[...]
    item_lv[...] = jnp.dot(h_item, wlv[...], preferred_element_type=jnp.float32) + blv[...]


def _post(out_item, out_user, p):
    blk = 1000
    grid = (N // blk,)
    row = pl.BlockSpec((blk, D), lambda i: (i, 0))
    full = pl.BlockSpec((D, D), lambda i: (0, 0))
    vec = pl.BlockSpec((1, D), lambda i: (0, 0))
    return pl.pallas_call(
        _post_body,
        grid=grid,
        in_specs=[row, row, vec, vec, full, vec, full, vec],
        out_specs=[row, row, row, row],
        out_shape=[jax.ShapeDtypeStruct((N, D), jnp.float32)] * 4,
    )(out_item, out_user,
      p['bias_u2i'].reshape(1, D), p['bias_i2u'].reshape(1, D),
      p['node_mu_W'], p['node_mu_b'].reshape(1, D),
      p['node_logvar_W'], p['node_logvar_b'].reshape(1, D))


def _edge_body(hl_hbm, hr_hbm, src_hbm, dst_hbm, att_hbm, out_hbm,
               srcc0, srcc1, dstc0, dstc1, ex_v, sg_v, rowA0, rowA1, rowB,
               att_v, zb_v, s_sh, out_sh, semA0, semA1, semB):
    c = lax.axis_index("c")
    w = lax.axis_index("s")
    base = c * E + w * EPT
    voff = pl.multiple_of(c * N, 8)
    hlv = hl_hbm.at[pl.ds(voff, N)]
    hrv = hr_hbm.at[pl.ds(voff, N)]

    zv = jnp.zeros((16,), jnp.float32)

    def _zero_zb(i, _):
        zb_v[pl.ds(i * 16, 16)] = zv
        return 0
    lax.fori_loop(0, 40, _zero_zb, 0)

    def _zero_rowA(i, _):
        r = i // 8
        k = i % 8
        rowA0[r, pl.ds(k * 16, 16)] = zv
        return 0
    lax.fori_loop(0, CHUNK * 8, _zero_rowA, 0)

    pltpu.sync_copy(zb_v, s_sh.at[pl.ds(w * 640, 640)])

    rstart = w * 624
    nrow16 = jnp.where(w == NT - 1, 40, 39)

    def _zero_out(i, _):
        pltpu.sync_copy(rowA0.at[pl.ds(0, 16)], out_sh.at[pl.ds(rstart + i * 16, 16)])
        return 0
    lax.fori_loop(0, nrow16, _zero_out, 0)

    pltpu.sync_copy(att_hbm.at[pl.ds(c * D, D)], att_v)

    plsc.subcore_barrier()

    att_regs = [att_v[pl.ds(k * 16, 16)] for k in range(8)]
    iota16 = lax.iota(jnp.int32, 16)

    srccs = (srcc0, srcc1)
    dstcs = (dstc0, dstc1)
    rowAs = (rowA0, rowA1)
    semAs = (semA0, semA1)

    def _stage(n, p):
        off = base + n * CHUNK
        pltpu.sync_copy(src_hbm.at[pl.ds(off, CHUNK)], srccs[p])
        pltpu.sync_copy(dst_hbm.at[pl.ds(off, CHUNK)], dstcs[p])

    def _issueA(p):
        pltpu.async_copy(hlv.at[srccs[p]], rowAs[p], semAs[p])

    def _waitA(p):
        pltpu.make_async_copy(hlv.at[srccs[p]], rowAs[p], semAs[p]).wait()

    def _issueB(p):
        pltpu.async_copy(hrv.at[dstcs[p]], rowB, semB)

    def _waitB(p):
        pltpu.make_async_copy(hrv.at[dstcs[p]], rowB, semB).wait()

    def _issueS(p):
        pltpu.async_copy(s_sh.at[dstcs[p]], sg_v, semB)

    def _waitS(p):
        pltpu.make_async_copy(s_sh.at[dstcs[p]], sg_v, semB).wait()

    def _compute1(n, p):
        _waitA(p)
        _waitB(p)
        ebase = n * CHUNK
        rA = rowAs[p]

        def _group(g, _):
            gb = g * 16
            evec = jnp.zeros((16,), jnp.float32)
            for e in range(16):
                acc = jnp.zeros((16,), jnp.float32)
                for k in range(8):
                    sl = pl.ds(k * 16, 16)
                    t = rA[gb + e, sl] + rowB[gb + e, sl]
                    z = jnp.maximum(t, 0.2 * t)
                    acc = acc + z * att_regs[k]
                lanes = [acc[i] for i in range(16)]
                while len(lanes) > 1:
                    lanes = [lanes[i] + lanes[i + 1] for i in range(0, len(lanes), 2)]
                evec = jnp.where(iota16 == e, lanes[0], evec)
            ex_v[pl.ds(ebase + gb, 16)] = jnp.exp(evec)
            return 0
        lax.fori_loop(0, CHUNK // 16, _group, 0)

        pltpu.sync_copy(ex_v.at[pl.ds(ebase, CHUNK)], s_sh.at[dstcs[p]], add=True)

    _sc1 = jax.named_scope("edge_phase1")
    _sc1.__enter__()
    _stage(0, 0)
    _issueA(0)
    _issueB(0)

    def _p1(j, _):
        a = 2 * j + 1
        b = 2 * j + 2
        _stage(a, 1)
        _issueA(1)
        _compute1(2 * j, 0)
        _issueB(1)
        _stage(b, 0)
        _issueA(0)
        _compute1(a, 1)
        _issueB(0)
        return 0
    lax.fori_loop(0, (NCHUNK - 1) // 2, _p1, 0)
    _compute1(NCHUNK - 1, 0)
    _sc1.__exit__(None, None, None)

    plsc.subcore_barrier()

    def _compute2(n, p):
        _waitA(p)
        _waitS(p)
        ebase = n * CHUNK
        rA = rowAs[p]

        def _alpha(g, _):
            sl = pl.ds(ebase + g * 16, 16)
            sv = sg_v[pl.ds(g * 16, 16)]
            ex_v[sl] = ex_v[sl] / (sv + 1e-16)
            return 0
        lax.fori_loop(0, CHUNK // 16, _alpha, 0)

        def _scale(g, _):
            gb = g * 16
            av16 = ex_v[pl.ds(ebase + gb, 16)]
            for e in range(16):
                av = jnp.broadcast_to(av16[e], (16,))
                for k in range(8):
                    sl = pl.ds(k * 16, 16)
                    rA[gb + e, sl] = rA[gb + e, sl] * av
            return 0
        lax.fori_loop(0, CHUNK // 16, _scale, 0)

        pltpu.sync_copy(rA, out_sh.at[dstcs[p]], add=True)

    _sc2 = jax.named_scope("edge_phase2")
    _sc2.__enter__()
    _stage(0, 0)
    _issueA(0)
    _issueS(0)

    def _p2(j, _):
        a = 2 * j + 1
        b = 2 * j + 2
        _stage(a, 1)
        _issueA(1)
        _compute2(2 * j, 0)
        _issueS(1)
        _stage(b, 0)
        _issueA(0)
        _compute2(a, 1)
        _issueS(0)
        return 0
    lax.fori_loop(0, (NCHUNK - 1) // 2, _p2, 0)
    _compute2(NCHUNK - 1, 0)
    _sc2.__exit__(None, None, None)

    plsc.subcore_barrier()

    def _wb(i, _):
        r = rstart + i * 16
        pltpu.sync_copy(out_sh.at[pl.ds(r, 16)], rowA0.at[pl.ds(0, 16)])
        pltpu.sync_copy(rowA0.at[pl.ds(0, 16)], out_hbm.at[c, pl.ds(r, 16)])
        return 0
    lax.fori_loop(0, nrow16, _wb, 0)


def _edge_stage(hl_all, hr_all, src, dst, att):
    mesh = plsc.VectorSubcoreMesh(core_axis_name="c", subcore_axis_name="s")
    f = pl.kernel(
        _edge_body,
        out_type=jax.ShapeDtypeStruct((2, N, D), jnp.float32),
        mesh=mesh,
        scratch_types=[
            pltpu.VMEM((CHUNK,), jnp.int32),
            pltpu.VMEM((CHUNK,), jnp.int32),
            pltpu.VMEM((CHUNK,), jnp.int32),
            pltpu.VMEM((CHUNK,), jnp.int32),
            pltpu.VMEM((EPT,), jnp.float32),
            pltpu.VMEM((CHUNK,), jnp.float32),
            pltpu.VMEM((CHUNK, D), jnp.float32),
            pltpu.VMEM((CHUNK, D), jnp.float32),
            pltpu.VMEM((CHUNK, D), jnp.float32),
            pltpu.VMEM((D,), jnp.float32),
            pltpu.VMEM((640,), jnp.float32),
            pltpu.VMEM_SHARED((N_S,), jnp.float32),
            pltpu.VMEM_SHARED((N, D), jnp.float32),
            pltpu.SemaphoreType.DMA,
            pltpu.SemaphoreType.DMA,
            pltpu.SemaphoreType.DMA,
        ],
    )
    return f(hl_all, hr_all, src, dst, att)


def kernel(x_user, x_item, edge_index_u2i, edge_index_i2u, params):
    p = params
    kn = jax.random.split(jax.random.key(42), 3)
    n0 = jax.random.normal(kn[0], x_user.shape, dtype=jnp.float32)
    n1 = jax.random.normal(kn[1], x_item.shape, dtype=jnp.float32)
    n2 = jax.random.normal(kn[2], (N_USER + N_ITEM, D), dtype=jnp.float32)

    hl_u2i, hr_u2i, hl_i2u, hr_i2u = _dense_pre(x_user, x_item, n0, n1, p)

    hl_all = jnp.concatenate([hl_u2i, hl_i2u], axis=0)
    hr_all = jnp.concatenate([hr_u2i, hr_i2u], axis=0)
    src = jnp.concatenate([edge_index_u2i[0], edge_index_i2u[0]])
    dst = jnp.concatenate([edge_index_u2i[1], edge_index_i2u[1]])
    att = jnp.concatenate([p['att_u2i'], p['att_i2u']])

    out = _edge_stage(hl_all, hr_all, src, dst, att)
    out_item, out_user = out[0], out[1]

    user_mu, user_lv, item_mu, item_lv = _post(out_item, out_user, p)

    x_hat = jnp.concatenate([x_user, x_item], axis=0) + n2
    a_mu, a_lv = _attr_branch(x_hat, p)

    return (user_mu, user_lv, item_mu, item_lv, a_mu, a_lv)


# speedup vs baseline: 1.5067x; 1.0029x over previous
import functools

import jax
import jax.numpy as jnp
from jax import lax
from jax.experimental import pallas as pl
from jax.experimental.pallas import tpu as pltpu
from jax.experimental.pallas import tpu_sc as plsc

N_USER = 10000
N_ITEM = 10000
N = 10000
D = 128
E = 160000

NT = 16
EPT = E // NT
CHUNK = 80
NCHUNK = EPT // CHUNK
ROWS_PT = N // NT
N_S = 10240
NCHUNK_PAD = 128
EPT_PAD = NCHUNK_PAD * CHUNK


def _pre_body(xu, xi, n0, n1, wlu, wru, wli, wri, blu, bru, bli, bri,
              hl_u2i, hr_u2i, hl_i2u, hr_i2u):
    a = xu[...] + n0[...]
    b = xi[...] + n1[...]
    hl_u2i[...] = jnp.dot(a, wlu[...], preferred_element_type=jnp.float32) + blu[...]
    hr_u2i[...] = jnp.dot(b, wru[...], preferred_element_type=jnp.float32) + bru[...]
    hl_i2u[...] = jnp.dot(b, wli[...], preferred_element_type=jnp.float32) + bli[...]
    hr_i2u[...] = jnp.dot(a, wri[...], preferred_element_type=jnp.float32) + bri[...]


def _dense_pre(xu, xi, n0, n1, p):
    blk = 1000
    grid = (N // blk,)
    row = pl.BlockSpec((blk, D), lambda i: (i, 0))
    full = pl.BlockSpec((D, D), lambda i: (0, 0))
    vec = pl.BlockSpec((1, D), lambda i: (0, 0))
    return pl.pallas_call(
        _pre_body,
        grid=grid,
        in_specs=[row, row, row, row, full, full, full, full, vec, vec, vec, vec],
        out_specs=[row, row, row, row],
        out_shape=[jax.ShapeDtypeStruct((N, D), jnp.float32)] * 4,
    )(xu, xi, n0, n1,
      p['Wl_u2i'], p['Wr_u2i'], p['Wl_i2u'], p['Wr_i2u'],
      p['bl_u2i'].reshape(1, D), p['br_u2i'].reshape(1, D),
      p['bl_i2u'].reshape(1, D), p['br_i2u'].reshape(1, D))


def _attr_body(xh, w1, b1, g1, be1, w2, b2, g2, be2, w3, b3, g3, be3,
               wmu, bmu, wlv, blv, a_mu, a_lv):
    h = xh[...]
    for (w, b, g, be) in ((w1, b1, g1, be1), (w2, b2, g2, be2), (w3, b3, g3, be3)):
        h = jnp.tanh(jnp.dot(h, w[...], preferred_element_type=jnp.float32) + b[...])
        mean = jnp.mean(h, axis=0, keepdims=True)
        var = jnp.mean((h - mean) ** 2, axis=0, keepdims=True)
        h = (h - mean) / jnp.sqrt(var + 1e-5) * g[...] + be[...]
    a_mu[...] = jnp.dot(h, wmu[...], preferred_element_type=jnp.float32) + bmu[...]
    a_lv[...] = jnp.dot(h, wlv[...], preferred_element_type=jnp.float32) + blv[...]


def _attr_branch(x_hat, p):
    n = x_hat.shape[0]
    args = [x_hat]
    for j in range(1, 4):
        args += [p['W%d' % j], p['b%d' % j].reshape(1, D),
                 p['bn%d_g' % j].reshape(1, D), p['bn%d_b' % j].reshape(1, D)]
    args += [p['attr_mu_W'], p['attr_mu_b'].reshape(1, D),
             p['attr_logvar_W'], p['attr_logvar_b'].reshape(1, D)]
    return pl.pallas_call(
        _attr_body,
        out_shape=[jax.ShapeDtypeStruct((n, D), jnp.float32)] * 2,
    )(*args)


def _post_body(oi, ou, bi, bu, wmu, bmu, wlv, blv,
               user_mu, user_lv, item_mu, item_lv):
    h_item = jnp.maximum(oi[...] + bi[...], 0.0)
    h_user = jnp.maximum(ou[...] + bu[...], 0.0)
    user_mu[...] = jnp.dot(h_user, wmu[...], preferred_element_type=jnp.float32) + bmu[...]
    user_lv[...] = jnp.dot(h_user, wlv[...], preferred_element_type=jnp.float32) + blv[...]
    item_mu[...] = jnp.dot(h_item, wmu[...], preferred_element_type=jnp.float32) + bmu[...]
    item_lv[...] = jnp.dot(h_item, wlv[...], preferred_element_type=jnp.float32) + blv[...]


def _post(out_item, out_user, p):
    blk = 1000
    grid = (N // blk,)
    row = pl.BlockSpec((blk, D), lambda i: (i, 0))
    full = pl.BlockSpec((D, D), lambda i: (0, 0))
    vec = pl.BlockSpec((1, D), lambda i: (0, 0))
    return pl.pallas_call(
        _post_body,
        grid=grid,
        in_specs=[row, row, vec, vec, full, vec, full, vec],
        out_specs=[row, row, row, row],
        out_shape=[jax.ShapeDtypeStruct((N, D), jnp.float32)] * 4,
    )(out_item, out_user,
      p['bias_u2i'].reshape(1, D), p['bias_i2u'].reshape(1, D),
      p['node_mu_W'], p['node_mu_b'].reshape(1, D),
      p['node_logvar_W'], p['node_logvar_b'].reshape(1, D))


def _edge_body(hl_hbm, hr_hbm, src_hbm, dst_hbm, att_hbm, out_hbm,
               srcc0, srcc1, dstc0, dstc1, ex_v, sg_v, rowA0, rowA1, rowB,
               att_v, zb_v, s_sh, out_sh, semA0, semA1, semB):
    c = lax.axis_index("c")
    w = lax.axis_index("s")
    base = c * E + w * EPT
    voff = pl.multiple_of(c * N, 8)
    hlv = hl_hbm.at[pl.ds(voff, N)]
    hrv = hr_hbm.at[pl.ds(voff, N)]

    zv = jnp.zeros((16,), jnp.float32)

    def _zero_zb(i, _):
        zb_v[pl.ds(i * 16, 16)] = zv
        return 0
    lax.fori_loop(0, 40, _zero_zb, 0)

    def _zero_rowA(i, _):
        r = i // 8
        k = i % 8
        rowA0[r, pl.ds(k * 16, 16)] = zv
        return 0
    lax.fori_loop(0, CHUNK * 8, _zero_rowA, 0)

    pltpu.sync_copy(zb_v, s_sh.at[pl.ds(w * 640, 640)])

    rstart = w * 624
    nrow16 = jnp.where(w == NT - 1, 40, 39)

    def _zero_out(i, _):
        pltpu.sync_copy(rowA0.at[pl.ds(0, 16)], out_sh.at[pl.ds(rstart + i * 16, 16)])
        return 0
    lax.fori_loop(0, nrow16, _zero_out, 0)

    pltpu.sync_copy(att_hbm.at[pl.ds(c * D, D)], att_v)

    plsc.subcore_barrier()

    att_regs = [att_v[pl.ds(k * 16, 16)] for k in range(8)]
    iota16 = lax.iota(jnp.int32, 16)

    srccs = (srcc0, srcc1)
    dstcs = (dstc0, dstc1)
    rowAs = (rowA0, rowA1)
    semAs = (semA0, semA1)

    def _stage(n, p):
        off = base + n * CHUNK
        pltpu.sync_copy(src_hbm.at[pl.ds(off, CHUNK)], srccs[p])
        pltpu.sync_copy(dst_hbm.at[pl.ds(off, CHUNK)], dstcs[p])

    def _issueA(p):
        pltpu.async_copy(hlv.at[srccs[p]], rowAs[p], semAs[p])

    def _waitA(p):
        pltpu.make_async_copy(hlv.at[srccs[p]], rowAs[p], semAs[p]).wait()

    def _issueB(p):
        pltpu.async_copy(hrv.at[dstcs[p]], rowB, semB)

    def _waitB(p):
        pltpu.make_async_copy(hrv.at[dstcs[p]], rowB, semB).wait()

    def _issueS(p):
        pltpu.async_copy(s_sh.at[dstcs[p]], sg_v, semB)

    def _waitS(p):
        pltpu.make_async_copy(s_sh.at[dstcs[p]], sg_v, semB).wait()

    def _compute1(n, p):
        _waitA(p)
        _waitB(p)
        ebase = n * CHUNK
        rA = rowAs[p]

        def _group(g, _):
            gb = g * 16
            evec = jnp.zeros((16,), jnp.float32)
            for e in range(16):
                acc = jnp.zeros((16,), jnp.float32)
                for k in range(8):
                    sl = pl.ds(k * 16, 16)
                    t = rA[gb + e, sl] + rowB[gb + e, sl]
                    z = jnp.maximum(t, 0.2 * t)
                    acc = acc + z * att_regs[k]
                lanes = [acc[i] for i in range(16)]
                while len(lanes) > 1:
                    lanes = [lanes[i] + lanes[i + 1] for i in range(0, len(lanes), 2)]
                evec = jnp.where(iota16 == e, lanes[0], evec)
            ex_v[pl.ds(ebase + gb, 16)] = jnp.exp(evec)
            return 0
        lax.fori_loop(0, CHUNK // 16, _group, 0)

        pltpu.sync_copy(ex_v.at[pl.ds(ebase, CHUNK)], s_sh.at[dstcs[p]], add=True)

    _sc1 = jax.named_scope("edge_phase1")
    _sc1.__enter__()
    _stage(0, 0)
    _issueA(0)
    _issueB(0)

    def _p1(j, _):
        a = 2 * j + 1
        b = 2 * j + 2
        _stage(a, 1)
        _issueA(1)
        _compute1(2 * j, 0)
        _issueB(1)
        _stage(b, 0)
        _issueA(0)
        _compute1(a, 1)
        _issueB(0)
        return 0
    lax.fori_loop(0, (NCHUNK - 1) // 2, _p1, 0)
    _compute1(NCHUNK - 1, 0)
    _sc1.__exit__(None, None, None)

    plsc.subcore_barrier()

    def _compute2(n, p):
        _waitA(p)
        _waitS(p)
        ebase = n * CHUNK
        rA = rowAs[p]

        def _scale(g, _):
            gb = g * 16
            sv = sg_v[pl.ds(gb, 16)]
            av16 = ex_v[pl.ds(ebase + gb, 16)] / (sv + 1e-16)
            for e in range(16):
                av = jnp.broadcast_to(av16[e], (16,))
                for k in range(8):
                    sl = pl.ds(k * 16, 16)
                    rA[gb + e, sl] = rA[gb + e, sl] * av
            return 0
        lax.fori_loop(0, CHUNK // 16, _scale, 0)

        pltpu.sync_copy(rA, out_sh.at[dstcs[p]], add=True)

    _sc2 = jax.named_scope("edge_phase2")
    _sc2.__enter__()
    _stage(0, 0)
    _issueA(0)
    _issueS(0)

    def _p2(j, _):
        a = 2 * j + 1
        b = 2 * j + 2
        _stage(a, 1)
        _issueA(1)
        _compute2(2 * j, 0)
        _issueS(1)
        _stage(b, 0)
        _issueA(0)
        _compute2(a, 1)
        _issueS(0)
        return 0
    lax.fori_loop(0, (NCHUNK - 1) // 2, _p2, 0)
    _compute2(NCHUNK - 1, 0)
    _sc2.__exit__(None, None, None)

    plsc.subcore_barrier()

    def _wb(i, _):
        r = rstart + i * 16
        pltpu.sync_copy(out_sh.at[pl.ds(r, 16)], rowA0.at[pl.ds(0, 16)])
        pltpu.sync_copy(rowA0.at[pl.ds(0, 16)], out_hbm.at[c, pl.ds(r, 16)])
        return 0
    lax.fori_loop(0, nrow16, _wb, 0)


def _edge_stage(hl_all, hr_all, src, dst, att):
    mesh = plsc.VectorSubcoreMesh(core_axis_name="c", subcore_axis_name="s")
    f = pl.kernel(
        _edge_body,
        out_type=jax.ShapeDtypeStruct((2, N, D), jnp.float32),
        mesh=mesh,
        scratch_types=[
            pltpu.VMEM((CHUNK,), jnp.int32),
            pltpu.VMEM((CHUNK,), jnp.int32),
            pltpu.VMEM((CHUNK,), jnp.int32),
            pltpu.VMEM((CHUNK,), jnp.int32),
            pltpu.VMEM((EPT,), jnp.float32),
            pltpu.VMEM((CHUNK,), jnp.float32),
            pltpu.VMEM((CHUNK, D), jnp.float32),
            pltpu.VMEM((CHUNK, D), jnp.float32),
            pltpu.VMEM((CHUNK, D), jnp.float32),
            pltpu.VMEM((D,), jnp.float32),
            pltpu.VMEM((640,), jnp.float32),
            pltpu.VMEM_SHARED((N_S,), jnp.float32),
            pltpu.VMEM_SHARED((N, D), jnp.float32),
            pltpu.SemaphoreType.DMA,
            pltpu.SemaphoreType.DMA,
            pltpu.SemaphoreType.DMA,
        ],
    )
    return f(hl_all, hr_all, src, dst, att)


def kernel(x_user, x_item, edge_index_u2i, edge_index_i2u, params):
    p = params
    kn = jax.random.split(jax.random.key(42), 3)
    n0 = jax.random.normal(kn[0], x_user.shape, dtype=jnp.float32)
    n1 = jax.random.normal(kn[1], x_item.shape, dtype=jnp.float32)
    n2 = jax.random.normal(kn[2], (N_USER + N_ITEM, D), dtype=jnp.float32)

    hl_u2i, hr_u2i, hl_i2u, hr_i2u = _dense_pre(x_user, x_item, n0, n1, p)

    hl_all = jnp.concatenate([hl_u2i, hl_i2u], axis=0)
    hr_all = jnp.concatenate([hr_u2i, hr_i2u], axis=0)
    src = jnp.concatenate([edge_index_u2i[0], edge_index_i2u[0]])
    dst = jnp.concatenate([edge_index_u2i[1], edge_index_i2u[1]])
    att = jnp.concatenate([p['att_u2i'], p['att_i2u']])

    out = _edge_stage(hl_all, hr_all, src, dst, att)
    out_item, out_user = out[0], out[1]

    user_mu, user_lv, item_mu, item_lv = _post(out_item, out_user, p)

    x_hat = jnp.concatenate([x_user, x_item], axis=0) + n2
    a_mu, a_lv = _attr_branch(x_hat, p)

    return (user_mu, user_lv, item_mu, item_lv, a_mu, a_lv)


# speedup vs baseline: 1.5077x; 1.0007x over previous
import jax
import jax.numpy as jnp
from jax import lax
from jax.experimental import pallas as pl
from jax.experimental.pallas import tpu as pltpu
from jax.experimental.pallas import tpu_sc as plsc

N_USER = 10000
N_ITEM = 10000
N = 10000
D = 128
E = 160000

NT = 16
EPT = E // NT
CHUNK = 80
NCHUNK = EPT // CHUNK
N_S = 10240


def _pre_body(xu, xi, n0, n1, wlu, wru, wli, wri, blu, bru, bli, bri,
              hl_u2i, hr_u2i, hl_i2u, hr_i2u):
    a = xu[...] + n0[...]
    b = xi[...] + n1[...]
    hl_u2i[...] = jnp.dot(a, wlu[...], preferred_element_type=jnp.float32) + blu[...]
    hr_u2i[...] = jnp.dot(b, wru[...], preferred_element_type=jnp.float32) + bru[...]
    hl_i2u[...] = jnp.dot(b, wli[...], preferred_element_type=jnp.float32) + bli[...]
    hr_i2u[...] = jnp.dot(a, wri[...], preferred_element_type=jnp.float32) + bri[...]


def _dense_pre(xu, xi, n0, n1, p):
    blk = 1000
    grid = (N // blk,)
    row = pl.BlockSpec((blk, D), lambda i: (i, 0))
    full = pl.BlockSpec((D, D), lambda i: (0, 0))
    vec = pl.BlockSpec((1, D), lambda i: (0, 0))
    return pl.pallas_call(
        _pre_body,
        grid=grid,
        in_specs=[row, row, row, row, full, full, full, full, vec, vec, vec, vec],
        out_specs=[row, row, row, row],
        out_shape=[jax.ShapeDtypeStruct((N, D), jnp.float32)] * 4,
    )(xu, xi, n0, n1,
      p['Wl_u2i'], p['Wr_u2i'], p['Wl_i2u'], p['Wr_i2u'],
      p['bl_u2i'].reshape(1, D), p['br_u2i'].reshape(1, D),
      p['bl_i2u'].reshape(1, D), p['br_i2u'].reshape(1, D))


def _attr_body(xh, w1, b1, g1, be1, w2, b2, g2, be2, w3, b3, g3, be3,
               wmu, bmu, wlv, blv, a_mu, a_lv):
    h = xh[...]
    for (w, b, g, be) in ((w1, b1, g1, be1), (w2, b2, g2, be2), (w3, b3, g3, be3)):
        h = jnp.tanh(jnp.dot(h, w[...], preferred_element_type=jnp.float32) + b[...])
        mean = jnp.mean(h, axis=0, keepdims=True)
        var = jnp.mean((h - mean) ** 2, axis=0, keepdims=True)
        h = (h - mean) / jnp.sqrt(var + 1e-5) * g[...] + be[...]
    a_mu[...] = jnp.dot(h, wmu[...], preferred_element_type=jnp.float32) + bmu[...]
    a_lv[...] = jnp.dot(h, wlv[...], preferred_element_type=jnp.float32) + blv[...]


def _attr_branch(x_hat, p):
    n = x_hat.shape[0]
    args = [x_hat]
    for j in range(1, 4):
        args += [p['W%d' % j], p['b%d' % j].reshape(1, D),
                 p['bn%d_g' % j].reshape(1, D), p['bn%d_b' % j].reshape(1, D)]
    args += [p['attr_mu_W'], p['attr_mu_b'].reshape(1, D),
             p['attr_logvar_W'], p['attr_logvar_b'].reshape(1, D)]
    return pl.pallas_call(
        _attr_body,
        out_shape=[jax.ShapeDtypeStruct((n, D), jnp.float32)] * 2,
    )(*args)


def _post_body(oi, ou, bi, bu, wmu, bmu, wlv, blv,
               user_mu, user_lv, item_mu, item_lv):
    h_item = jnp.maximum(oi[...] + bi[...], 0.0)
    h_user = jnp.maximum(ou[...] + bu[...], 0.0)
    user_mu[...] = jnp.dot(h_user, wmu[...], preferred_element_type=jnp.float32) + bmu[...]
    user_lv[...] = jnp.dot(h_user, wlv[...], preferred_element_type=jnp.float32) + blv[...]
    item_mu[...] = jnp.dot(h_item, wmu[...], preferred_element_type=jnp.float32) + bmu[...]
    item_lv[...] = jnp.dot(h_item, wlv[...], preferred_element_type=jnp.float32) + blv[...]


def _post(out_item, out_user, p):
    blk = 1000
    grid = (N // blk,)
    row = pl.BlockSpec((blk, D), lambda i: (i, 0))
    full = pl.BlockSpec((D, D), lambda i: (0, 0))
    vec = pl.BlockSpec((1, D), lambda i: (0, 0))
    return pl.pallas_call(
        _post_body,
        grid=grid,
        in_specs=[row, row, vec, vec, full, vec, full, vec],
        out_specs=[row, row, row, row],
        out_shape=[jax.ShapeDtypeStruct((N, D), jnp.float32)] * 4,
    )(out_item, out_user,
      p['bias_u2i'].reshape(1, D), p['bias_i2u'].reshape(1, D),
      p['node_mu_W'], p['node_mu_b'].reshape(1, D),
      p['node_logvar_W'], p['node_logvar_b'].reshape(1, D))


def _edge_body(hl_hbm, hr_hbm, src_hbm, dst_hbm, att_hbm, out_hbm,
               srcc0, srcc1, dstc0, dstc1, ex_v, sg_v, rowA0, rowA1, rowB,
               att_v, zb_v, s_sh, out_sh, semA0, semA1, semB):
    c = lax.axis_index("c")
    w = lax.axis_index("s")
    base = c * E + w * EPT
    voff = pl.multiple_of(c * N, 8)
    hlv = hl_hbm.at[pl.ds(voff, N)]
    hrv = hr_hbm.at[pl.ds(voff, N)]

    zv = jnp.zeros((16,), jnp.float32)

    def _zero_zb(i, _):
        zb_v[pl.ds(i * 16, 16)] = zv
        return 0
    lax.fori_loop(0, 40, _zero_zb, 0)

    def _zero_rowA(i, _):
        r = i // 8
        k = i % 8
        rowA0[r, pl.ds(k * 16, 16)] = zv
        return 0
    lax.fori_loop(0, CHUNK * 8, _zero_rowA, 0)

    pltpu.sync_copy(zb_v, s_sh.at[pl.ds(w * 640, 640)])

    rstart = w * 624
    nrow16 = jnp.where(w == NT - 1, 40, 39)

    def _zero_out(i, _):
        pltpu.sync_copy(rowA0.at[pl.ds(0, 16)], out_sh.at[pl.ds(rstart + i * 16, 16)])
        return 0
    lax.fori_loop(0, nrow16, _zero_out, 0)

    pltpu.sync_copy(att_hbm.at[pl.ds(c * D, D)], att_v)

    plsc.subcore_barrier()

    att_regs = [att_v[pl.ds(k * 16, 16)] for k in range(8)]
    iota16 = lax.iota(jnp.int32, 16)

    srccs = (srcc0, srcc1)
    dstcs = (dstc0, dstc1)
    rowAs = (rowA0, rowA1)
    semAs = (semA0, semA1)

    def _stage(n, p):
        off = base + n * CHUNK
        pltpu.sync_copy(src_hbm.at[pl.ds(off, CHUNK)], srccs[p])
        pltpu.sync_copy(dst_hbm.at[pl.ds(off, CHUNK)], dstcs[p])

    def _issueA(p):
        pltpu.async_copy(hlv.at[srccs[p]], rowAs[p], semAs[p])

    def _waitA(p):
        pltpu.make_async_copy(hlv.at[srccs[p]], rowAs[p], semAs[p]).wait()

    def _issueB(p):
        pltpu.async_copy(hrv.at[dstcs[p]], rowB, semB)

    def _waitB(p):
        pltpu.make_async_copy(hrv.at[dstcs[p]], rowB, semB).wait()

    def _issueS(p):
        pltpu.async_copy(s_sh.at[dstcs[p]], sg_v, semB)

    def _waitS(p):
        pltpu.make_async_copy(s_sh.at[dstcs[p]], sg_v, semB).wait()

    def _compute1(n, p):
        _waitA(p)
        _waitB(p)
        ebase = n * CHUNK
        rA = rowAs[p]

        def _group(g, _):
            gb = g * 16
            evec = jnp.zeros((16,), jnp.float32)
            for e in range(16):
                acc = jnp.zeros((16,), jnp.float32)
                for k in range(8):
                    sl = pl.ds(k * 16, 16)
                    t = rA[gb + e, sl] + rowB[gb + e, sl]
                    z = jnp.maximum(t, 0.2 * t)
                    acc = acc + z * att_regs[k]
                lanes = [acc[i] for i in range(16)]
                while len(lanes) > 1:
                    lanes = [lanes[i] + lanes[i + 1] for i in range(0, len(lanes), 2)]
                evec = jnp.where(iota16 == e, lanes[0], evec)
            ex_v[pl.ds(ebase + gb, 16)] = jnp.exp(evec)
            return 0
        lax.fori_loop(0, CHUNK // 16, _group, 0)

        pltpu.sync_copy(ex_v.at[pl.ds(ebase, CHUNK)], s_sh.at[dstcs[p]], add=True)

    _sc1 = jax.named_scope("edge_phase1")
    _sc1.__enter__()
    _stage(0, 0)
    _issueA(0)
    _issueB(0)

    def _p1(j, _):
        a = 2 * j + 1
        b = 2 * j + 2
        _stage(a, 1)
        _issueA(1)
        _compute1(2 * j, 0)
        _issueB(1)
        _stage(b, 0)
        _issueA(0)
        _compute1(a, 1)
        _issueB(0)
        return 0
    lax.fori_loop(0, (NCHUNK - 1) // 2, _p1, 0)
    _compute1(NCHUNK - 1, 0)
    _sc1.__exit__(None, None, None)

    plsc.subcore_barrier()

    def _compute2(n, p):
        _waitA(p)
        _waitS(p)
        ebase = n * CHUNK
        rA = rowAs[p]

        def _scale(g, _):
            gb = g * 16
            sv = sg_v[pl.ds(gb, 16)]
            av16 = ex_v[pl.ds(ebase + gb, 16)] / (sv + 1e-16)
            for e in range(16):
                av = jnp.broadcast_to(av16[e], (16,))
                for k in range(8):
                    sl = pl.ds(k * 16, 16)
                    rA[gb + e, sl] = rA[gb + e, sl] * av
            return 0
        lax.fori_loop(0, CHUNK // 16, _scale, 0)

        pltpu.sync_copy(rA, out_sh.at[dstcs[p]], add=True)

    _sc2 = jax.named_scope("edge_phase2")
    _sc2.__enter__()
    _stage(0, 0)
    _issueA(0)
    _issueS(0)

    def _p2(j, _):
        a = 2 * j + 1
        b = 2 * j + 2
        _stage(a, 1)
        _issueA(1)
        _compute2(2 * j, 0)
        _issueS(1)
        _stage(b, 0)
        _issueA(0)
        _compute2(a, 1)
        _issueS(0)
        return 0
    lax.fori_loop(0, (NCHUNK - 1) // 2, _p2, 0)
    _compute2(NCHUNK - 1, 0)
    _sc2.__exit__(None, None, None)

    plsc.subcore_barrier()

    def _wb(i, _):
        r = rstart + i * 16
        pltpu.sync_copy(out_sh.at[pl.ds(r, 16)], rowA0.at[pl.ds(0, 16)])
        pltpu.sync_copy(rowA0.at[pl.ds(0, 16)], out_hbm.at[c, pl.ds(r, 16)])
        return 0
    lax.fori_loop(0, nrow16, _wb, 0)


def _edge_stage(hl_all, hr_all, src, dst, att):
    mesh = plsc.VectorSubcoreMesh(core_axis_name="c", subcore_axis_name="s")
    f = pl.kernel(
        _edge_body,
        out_type=jax.ShapeDtypeStruct((2, N, D), jnp.float32),
        mesh=mesh,
        scratch_types=[
            pltpu.VMEM((CHUNK,), jnp.int32),
            pltpu.VMEM((CHUNK,), jnp.int32),
            pltpu.VMEM((CHUNK,), jnp.int32),
            pltpu.VMEM((CHUNK,), jnp.int32),
            pltpu.VMEM((EPT,), jnp.float32),
            pltpu.VMEM((CHUNK,), jnp.float32),
            pltpu.VMEM((CHUNK, D), jnp.float32),
            pltpu.VMEM((CHUNK, D), jnp.float32),
            pltpu.VMEM((CHUNK, D), jnp.float32),
            pltpu.VMEM((D,), jnp.float32),
            pltpu.VMEM((640,), jnp.float32),
            pltpu.VMEM_SHARED((N_S,), jnp.float32),
            pltpu.VMEM_SHARED((N, D), jnp.float32),
            pltpu.SemaphoreType.DMA,
            pltpu.SemaphoreType.DMA,
            pltpu.SemaphoreType.DMA,
        ],
    )
    return f(hl_all, hr_all, src, dst, att)


def kernel(x_user, x_item, edge_index_u2i, edge_index_i2u, params):
    p = params
    kn = jax.random.split(jax.random.key(42), 3)
    n0 = jax.random.normal(kn[0], x_user.shape, dtype=jnp.float32)
    n1 = jax.random.normal(kn[1], x_item.shape, dtype=jnp.float32)
    n2 = jax.random.normal(kn[2], (N_USER + N_ITEM, D), dtype=jnp.float32)

    hl_u2i, hr_u2i, hl_i2u, hr_i2u = _dense_pre(x_user, x_item, n0, n1, p)

    hl_all = jnp.concatenate([hl_u2i, hl_i2u], axis=0)
    hr_all = jnp.concatenate([hr_u2i, hr_i2u], axis=0)
    src = jnp.concatenate([edge_index_u2i[0], edge_index_i2u[0]])
    dst = jnp.concatenate([edge_index_u2i[1], edge_index_i2u[1]])
    att = jnp.concatenate([p['att_u2i'], p['att_i2u']])

    out = _edge_stage(hl_all, hr_all, src, dst, att)
    out_item, out_user = out[0], out[1]

    user_mu, user_lv, item_mu, item_lv = _post(out_item, out_user, p)

    x_hat = jnp.concatenate([x_user, x_item], axis=0) + n2
    a_mu, a_lv = _attr_branch(x_hat, p)

    return (user_mu, user_lv, item_mu, item_lv, a_mu, a_lv)
